# baseline (speedup 1.0000x reference)
import functools
import jax
import jax.numpy as jnp
from jax import lax
from jax.experimental import pallas as pl
from jax.experimental.pallas import tpu as pltpu
from jax.experimental.pallas import tpu_sc as plsc

_B, _K, _NC, _NS = 8, 2, 16, 2048
_NZ, _NX = 256, 128
_NP = _NZ * _NX
_NW = 32
_TPB = _NW // _B
_PPT = _NP // _TPB
_CP = 1024
_ZC = _CP // _NX
_NJ = _PPT // _CP
_PLANE = _NC * _NS


def _tc_pack(rfs2d):
    def body(x_ref, o_ref):
        x = x_ref[...]
        d = pltpu.roll(x, x.shape[1] - 1, 1) - x
        lo = jax.lax.bitcast_convert_type(
            x.astype(jnp.bfloat16), jnp.uint16).astype(jnp.uint32)
        hi = jax.lax.bitcast_convert_type(
            d.astype(jnp.bfloat16), jnp.uint16).astype(jnp.uint32)
        o_ref[...] = jax.lax.bitcast_convert_type(lo | (hi << 16), jnp.int32)

    return pl.pallas_call(
        body,
        out_shape=jax.ShapeDtypeStruct(rfs2d.shape, jnp.int32),
    )(rfs2d)


def _sc_das(rfs, samples_3d, rowidx):
    mesh = plsc.VectorSubcoreMesh(core_axis_name="core", subcore_axis_name="subcore")

    @functools.partial(
        pl.kernel,
        mesh=mesh,
        out_type=jax.ShapeDtypeStruct((_B, _NZ, _K, _NX), jnp.float32),
        compiler_params=pltpu.CompilerParams(needs_layout_passes=False),
        scratch_types=[
            pltpu.VMEM((_PLANE,), jnp.int32),
            pltpu.VMEM((_PLANE,), jnp.int32),
            pltpu.VMEM((2, _NC, _ZC, _NX), jnp.float32),
            pltpu.VMEM((2, _ZC, _K, _NX), jnp.float32),
            pltpu.VMEM((_NC,), jnp.int32),
            pltpu.SemaphoreType.DMA,
            pltpu.SemaphoreType.DMA,
            pltpu.SemaphoreType.DMA,
            pltpu.SemaphoreType.DMA,
            pltpu.SemaphoreType.DMA,
        ],
    )
    def k(rfs_hbm, samples_hbm, rowidx_hbm, out_hbm, rfs0_v, rfs1_v, idx_v, acc_v,
          row_v, sem_r, sem_i0, sem_i1, sem_o0, sem_o1):
        cid = lax.axis_index("core")
        sid = lax.axis_index("subcore")
        wid = sid * 2 + cid
        b = wid // _TPB
        q = wid % _TPB
        zbase = q * (_NJ * _ZC)

        copies = []
        for kk, dst in ((0, rfs0_v), (1, rfs1_v)):
            for c in range(_NC):
                copies.append(pltpu.async_copy(
                    rfs_hbm.at[b, kk, c], dst.at[pl.ds(c * _NS, _NS)], sem_r))
        pltpu.sync_copy(rowidx_hbm.at[wid], row_v)
        pltpu.async_copy(
            samples_hbm.at[row_v, pl.ds(zbase, _ZC)], idx_v.at[0], sem_i0)
        for cp in copies:
            cp.wait()

        def idx_copy(j, buf, sem):
            z0 = zbase + j * _ZC
            return pltpu.make_async_copy(
                samples_hbm.at[row_v, pl.ds(z0, _ZC)], idx_v.at[buf], sem)

        def out_copy(j, buf, sem):
            z0 = zbase + j * _ZC
            return pltpu.make_async_copy(
                acc_v.at[buf], out_hbm.at[b, pl.ds(z0, _ZC)], sem)

        def phase(j, buf, sem_i, sem_other, sem_o):
            idx_copy(j, buf, sem_i).wait()

            @pl.when(j + 1 < _NJ)
            def _():
                idx_copy(j + 1, 1 - buf, sem_other).start()

            @pl.when(j >= 2)
            def _():
                out_copy(j, buf, sem_o).wait()

            for cq in range(_NC // 4):
                @plsc.parallel_loop(0, _CP // 32)
                def gbody(g2):
                    zloc = g2 // 4
                    xb0 = (g2 % 4) * 32
                    for half in range(2):
                        xbase = xb0 + half * 16
                        acc0 = jnp.zeros((16,), jnp.float32)
                        acc1 = jnp.zeros((16,), jnp.float32)
                        for u in range(4):
                            c = cq * 4 + u
                            cb = c * _NS
                            fi = idx_v[buf, c, zloc, pl.ds(xbase, 16)]
                            i0 = fi.astype(jnp.int32)
                            w = fi - i0.astype(jnp.float32)
                            p0 = plsc.load_gather(rfs0_v.at[pl.ds(cb, _NS)], [i0])
                            p1 = plsc.load_gather(rfs1_v.at[pl.ds(cb, _NS)], [i0])
                            v00, d0 = plsc.unpack(
                                plsc.bitcast(p0, jnp.bfloat16),
                                format=plsc.PackFormat.INTERLEAVED)
                            v10, d1 = plsc.unpack(
                                plsc.bitcast(p1, jnp.bfloat16),
                                format=plsc.PackFormat.INTERLEAVED)
                            acc0 = acc0 + v00 + w * d0
                            acc1 = acc1 + v10 + w * d1
                        if cq == 0:
                            acc_v[buf, zloc, 0, pl.ds(xbase, 16)] = acc0
                            acc_v[buf, zloc, 1, pl.ds(xbase, 16)] = acc1
                        else:
                            plsc.addupdate(
                                acc_v.at[buf, zloc, 0, pl.ds(xbase, 16)], acc0)
                            plsc.addupdate(
                                acc_v.at[buf, zloc, 1, pl.ds(xbase, 16)], acc1)

            out_copy(j, buf, sem_o).start()

        def jbody(i, carry):
            jj = i * 2
            phase(jj, 0, sem_i0, sem_i1, sem_o0)
            phase(jj + 1, 1, sem_i1, sem_i0, sem_o1)
            return carry

        lax.fori_loop(0, _NJ // 2, jbody, 0)
        out_copy(_NJ - 2, 0, sem_o0).wait()
        out_copy(_NJ - 1, 1, sem_o1).wait()

    return k(rfs, samples_3d, rowidx)


def kernel(rfs, ids, samples_idx):
    b, kk, nc, ns = rfs.shape
    s, _, nz, nx = samples_idx.shape
    samples_3d = samples_idx.reshape(s * nc, nz, nx)
    packed = _tc_pack(rfs.reshape(b * kk * nc, ns)).reshape(b, kk, nc, ns)
    w = jnp.arange(_NW, dtype=jnp.int32)
    cc = jnp.arange(nc, dtype=jnp.int32)
    rows = ids[w // _TPB][:, None] * nc + cc[None, :]
    out = _sc_das(packed, samples_3d, rows.astype(jnp.int32))
    return out.transpose(0, 1, 3, 2)


# speedup vs baseline: 1.0842x; 1.0842x over previous
import functools
import jax
import jax.numpy as jnp
from jax import lax
from jax.experimental import pallas as pl
from jax.experimental.pallas import tpu as pltpu
from jax.experimental.pallas import tpu_sc as plsc

_B, _K, _NC, _NS = 8, 2, 16, 2048
_NZ, _NX = 256, 128
_NP = _NZ * _NX
_NW = 32
_TPB = _NW // _B
_PPT = _NP // _TPB
_CP = 1024
_ZC = _CP // _NX
_NJ = _PPT // _CP
_PLANE = _NC * _NS


def _tc_pack(rfs2d):
    def body(x_ref, o_ref):
        x = x_ref[...]
        d = pltpu.roll(x, x.shape[1] - 1, 1) - x
        lo = jax.lax.bitcast_convert_type(
            x.astype(jnp.bfloat16), jnp.uint16).astype(jnp.uint32)
        hi = jax.lax.bitcast_convert_type(
            d.astype(jnp.bfloat16), jnp.uint16).astype(jnp.uint32)
        o_ref[...] = jax.lax.bitcast_convert_type(lo | (hi << 16), jnp.int32)

    return pl.pallas_call(
        body,
        out_shape=jax.ShapeDtypeStruct(rfs2d.shape, jnp.int32),
    )(rfs2d)


def _sc_das(rfs, samples_3d, rowidx):
    mesh = plsc.VectorSubcoreMesh(core_axis_name="core", subcore_axis_name="subcore")

    @functools.partial(
        pl.kernel,
        mesh=mesh,
        out_type=jax.ShapeDtypeStruct((_B, _NZ, _K, _NX), jnp.float32),
        compiler_params=pltpu.CompilerParams(needs_layout_passes=False),
        scratch_types=[
            pltpu.VMEM((_PLANE,), jnp.int32),
            pltpu.VMEM((_PLANE,), jnp.int32),
            pltpu.VMEM((2, _NC, _ZC, _NX), jnp.float32),
            pltpu.VMEM((2, _ZC, _K, _NX), jnp.float32),
            pltpu.VMEM((_NC,), jnp.int32),
            pltpu.SemaphoreType.DMA,
            pltpu.SemaphoreType.DMA,
            pltpu.SemaphoreType.DMA,
            pltpu.SemaphoreType.DMA,
            pltpu.SemaphoreType.DMA,
        ],
    )
    def k(rfs_hbm, samples_hbm, rowidx_hbm, out_hbm, rfs0_v, rfs1_v, idx_v, acc_v,
          row_v, sem_r, sem_i0, sem_i1, sem_o0, sem_o1):
        cid = lax.axis_index("core")
        sid = lax.axis_index("subcore")
        wid = sid * 2 + cid
        b = wid // _TPB
        q = wid % _TPB
        zbase = q * (_NJ * _ZC)

        copies = []
        for kk, dst in ((0, rfs0_v), (1, rfs1_v)):
            for c in range(_NC):
                copies.append(pltpu.async_copy(
                    rfs_hbm.at[b, kk, c], dst.at[pl.ds(c * _NS, _NS)], sem_r))
        pltpu.sync_copy(rowidx_hbm.at[wid], row_v)
        pltpu.async_copy(
            samples_hbm.at[row_v, pl.ds(zbase, _ZC)], idx_v.at[0], sem_i0)
        for cp in copies:
            cp.wait()

        def idx_copy(j, buf, sem):
            z0 = zbase + j * _ZC
            return pltpu.make_async_copy(
                samples_hbm.at[row_v, pl.ds(z0, _ZC)], idx_v.at[buf], sem)

        def out_copy(j, buf, sem):
            z0 = zbase + j * _ZC
            return pltpu.make_async_copy(
                acc_v.at[buf], out_hbm.at[b, pl.ds(z0, _ZC)], sem)

        def phase(j, buf, sem_i, sem_other, sem_o):
            idx_copy(j, buf, sem_i).wait()

            @pl.when(j + 1 < _NJ)
            def _():
                idx_copy(j + 1, 1 - buf, sem_other).start()

            @pl.when(j >= 2)
            def _():
                out_copy(j, buf, sem_o).wait()

            for cq in range(_NC // 4):
                @plsc.parallel_loop(0, _CP // 16)
                def gbody(g):
                    zloc = g // 8
                    xbase = (g % 8) * 16
                    acc0 = jnp.zeros((16,), jnp.float32)
                    acc1 = jnp.zeros((16,), jnp.float32)
                    for u in range(4):
                        c = cq * 4 + u
                        cb = c * _NS
                        fi = idx_v[buf, c, zloc, pl.ds(xbase, 16)]
                        i0 = fi.astype(jnp.int32)
                        w = fi - i0.astype(jnp.float32)
                        p0 = plsc.load_gather(rfs0_v.at[pl.ds(cb, _NS)], [i0])
                        p1 = plsc.load_gather(rfs1_v.at[pl.ds(cb, _NS)], [i0])
                        v00, d0 = plsc.unpack(
                            plsc.bitcast(p0, jnp.bfloat16),
                            format=plsc.PackFormat.INTERLEAVED)
                        v10, d1 = plsc.unpack(
                            plsc.bitcast(p1, jnp.bfloat16),
                            format=plsc.PackFormat.INTERLEAVED)
                        acc0 = acc0 + v00 + w * d0
                        acc1 = acc1 + v10 + w * d1
                    if cq == 0:
                        acc_v[buf, zloc, 0, pl.ds(xbase, 16)] = acc0
                        acc_v[buf, zloc, 1, pl.ds(xbase, 16)] = acc1
                    else:
                        plsc.addupdate(acc_v.at[buf, zloc, 0, pl.ds(xbase, 16)], acc0)
                        plsc.addupdate(acc_v.at[buf, zloc, 1, pl.ds(xbase, 16)], acc1)

            out_copy(j, buf, sem_o).start()

        def jbody(i, carry):
            jj = i * 2
            phase(jj, 0, sem_i0, sem_i1, sem_o0)
            phase(jj + 1, 1, sem_i1, sem_i0, sem_o1)
            return carry

        lax.fori_loop(0, _NJ // 2, jbody, 0)
        out_copy(_NJ - 2, 0, sem_o0).wait()
        out_copy(_NJ - 1, 1, sem_o1).wait()

    return k(rfs, samples_3d, rowidx)


def kernel(rfs, ids, samples_idx):
    b, kk, nc, ns = rfs.shape
    s, _, nz, nx = samples_idx.shape
    samples_3d = samples_idx.reshape(s * nc, nz, nx)
    packed = _tc_pack(rfs.reshape(b * kk * nc, ns)).reshape(b, kk, nc, ns)
    w = jnp.arange(_NW, dtype=jnp.int32)
    cc = jnp.arange(nc, dtype=jnp.int32)
    rows = ids[w // _TPB][:, None] * nc + cc[None, :]
    out = _sc_das(packed, samples_3d, rows.astype(jnp.int32))
    return out.transpose(0, 1, 3, 2)


# speedup vs baseline: 1.0934x; 1.0085x over previous
import functools
import jax
import jax.numpy as jnp
from jax import lax
from jax.experimental import pallas as pl
from jax.experimental.pallas import tpu as pltpu
from jax.experimental.pallas import tpu_sc as plsc

_B, _K, _NC, _NS = 8, 2, 16, 2048
_NZ, _NX = 256, 128
_NP = _NZ * _NX
_NW = 32
_TPB = _NW // _B
_PPT = _NP // _TPB
_CP = 1024
_ZC = _CP // _NX
_NJ = _PPT // _CP
_PLANE = _NC * _NS


def _tc_pack(rfs2d):
    def body(x_ref, o_ref):
        x = x_ref[...]
        d = pltpu.roll(x, x.shape[1] - 1, 1) - x
        lo = jax.lax.bitcast_convert_type(
            x.astype(jnp.bfloat16), jnp.uint16).astype(jnp.uint32)
        hi = jax.lax.bitcast_convert_type(
            d.astype(jnp.bfloat16), jnp.uint16).astype(jnp.uint32)
        o_ref[...] = jax.lax.bitcast_convert_type(lo | (hi << 16), jnp.int32)

    return pl.pallas_call(
        body,
        out_shape=jax.ShapeDtypeStruct(rfs2d.shape, jnp.int32),
    )(rfs2d)


def _sc_das(rfs, samples_3d, rowidx):
    mesh = plsc.VectorSubcoreMesh(core_axis_name="core", subcore_axis_name="subcore")

    @functools.partial(
        pl.kernel,
        mesh=mesh,
        out_type=jax.ShapeDtypeStruct((_B, _NZ, _K, _NX), jnp.float32),
        compiler_params=pltpu.CompilerParams(needs_layout_passes=False),
        scratch_types=[
            pltpu.VMEM((_PLANE,), jnp.int32),
            pltpu.VMEM((_PLANE,), jnp.int32),
            pltpu.VMEM((2, _NC, _ZC, _NX), jnp.float32),
            pltpu.VMEM((2, _ZC, _K, _NX), jnp.float32),
            pltpu.VMEM((_NC,), jnp.int32),
            pltpu.SemaphoreType.DMA,
            pltpu.SemaphoreType.DMA,
            pltpu.SemaphoreType.DMA,
            pltpu.SemaphoreType.DMA,
            pltpu.SemaphoreType.DMA,
        ],
    )
    def k(rfs_hbm, samples_hbm, rowidx_hbm, out_hbm, rfs0_v, rfs1_v, idx_v, acc_v,
          row_v, sem_r, sem_i0, sem_i1, sem_o0, sem_o1):
        cid = lax.axis_index("core")
        sid = lax.axis_index("subcore")
        wid = sid * 2 + cid
        b = wid // _TPB
        q = wid % _TPB
        zbase = q * (_NJ * _ZC)

        copies = []
        for kk, dst in ((0, rfs0_v), (1, rfs1_v)):
            for c in range(_NC):
                copies.append(pltpu.async_copy(
                    rfs_hbm.at[b, kk, c], dst.at[pl.ds(c * _NS, _NS)], sem_r))
        pltpu.sync_copy(rowidx_hbm.at[wid], row_v)
        pltpu.async_copy(
            samples_hbm.at[row_v, pl.ds(zbase, _ZC)], idx_v.at[0], sem_i0)
        for cp in copies:
            cp.wait()

        def idx_copy(j, buf, sem):
            z0 = zbase + j * _ZC
            return pltpu.make_async_copy(
                samples_hbm.at[row_v, pl.ds(z0, _ZC)], idx_v.at[buf], sem)

        def out_copy(j, buf, sem):
            z0 = zbase + j * _ZC
            return pltpu.make_async_copy(
                acc_v.at[buf], out_hbm.at[b, pl.ds(z0, _ZC)], sem)

        def phase(j, buf, sem_i, sem_other, sem_o):
            idx_copy(j, buf, sem_i).wait()

            @pl.when(j + 1 < _NJ)
            def _():
                idx_copy(j + 1, 1 - buf, sem_other).start()

            @pl.when(j >= 2)
            def _():
                out_copy(j, buf, sem_o).wait()

            for cq in range(_NC // 8):
                @plsc.parallel_loop(0, _CP // 16)
                def gbody(g):
                    zloc = g // 8
                    xbase = (g % 8) * 16
                    acc0 = jnp.zeros((16,), jnp.float32)
                    acc1 = jnp.zeros((16,), jnp.float32)
                    for u in range(8):
                        c = cq * 8 + u
                        cb = c * _NS
                        fi = idx_v[buf, c, zloc, pl.ds(xbase, 16)]
                        i0 = fi.astype(jnp.int32)
                        w = fi - i0.astype(jnp.float32)
                        p0 = plsc.load_gather(rfs0_v.at[pl.ds(cb, _NS)], [i0])
                        p1 = plsc.load_gather(rfs1_v.at[pl.ds(cb, _NS)], [i0])
                        v00, d0 = plsc.unpack(
                            plsc.bitcast(p0, jnp.bfloat16),
                            format=plsc.PackFormat.INTERLEAVED)
                        v10, d1 = plsc.unpack(
                            plsc.bitcast(p1, jnp.bfloat16),
                            format=plsc.PackFormat.INTERLEAVED)
                        acc0 = acc0 + v00 + w * d0
                        acc1 = acc1 + v10 + w * d1
                    if cq == 0:
                        acc_v[buf, zloc, 0, pl.ds(xbase, 16)] = acc0
                        acc_v[buf, zloc, 1, pl.ds(xbase, 16)] = acc1
                    else:
                        plsc.addupdate(acc_v.at[buf, zloc, 0, pl.ds(xbase, 16)], acc0)
                        plsc.addupdate(acc_v.at[buf, zloc, 1, pl.ds(xbase, 16)], acc1)

            out_copy(j, buf, sem_o).start()

        def jbody(i, carry):
            jj = i * 2
            phase(jj, 0, sem_i0, sem_i1, sem_o0)
            phase(jj + 1, 1, sem_i1, sem_i0, sem_o1)
            return carry

        lax.fori_loop(0, _NJ // 2, jbody, 0)
        out_copy(_NJ - 2, 0, sem_o0).wait()
        out_copy(_NJ - 1, 1, sem_o1).wait()

    return k(rfs, samples_3d, rowidx)


def kernel(rfs, ids, samples_idx):
    b, kk, nc, ns = rfs.shape
    s, _, nz, nx = samples_idx.shape
    samples_3d = samples_idx.reshape(s * nc, nz, nx)
    packed = _tc_pack(rfs.reshape(b * kk * nc, ns)).reshape(b, kk, nc, ns)
    w = jnp.arange(_NW, dtype=jnp.int32)
    cc = jnp.arange(nc, dtype=jnp.int32)
    rows = ids[w // _TPB][:, None] * nc + cc[None, :]
    out = _sc_das(packed, samples_3d, rows.astype(jnp.int32))
    return out.transpose(0, 1, 3, 2)
